# baseline (speedup 1.0000x reference)
import jax
import jax.numpy as jnp
from jax.experimental import pallas as pl


def _pass1_kernel(L_ref, x0_ref, x1_ref):
    x1_ref[...] = jnp.dot(L_ref[...], x0_ref[...],
                          preferred_element_type=jnp.float32)


def _pass2_kernel(L_ref, x1_ref, x0b_ref, x1b_ref, WA_ref, WB_ref, WC_ref,
                  bias_ref, o_ref):
    t = jnp.dot(L_ref[...], x1_ref[...], preferred_element_type=jnp.float32)
    o_ref[...] = (
        jnp.dot(x0b_ref[...], WA_ref[...], preferred_element_type=jnp.float32)
        + jnp.dot(x1b_ref[...], WB_ref[...], preferred_element_type=jnp.float32)
        + jnp.dot(t, WC_ref[...], preferred_element_type=jnp.float32)
        + bias_ref[...]
    )


def kernel(x, L, weight, bias):
    N, M, Fin = x.shape
    Fout = weight.shape[1]
    x0 = jnp.transpose(x, (1, 0, 2)).reshape(M, N * Fin)

    W = weight.reshape(Fin, 3, Fout)
    eyeN = jnp.eye(N, dtype=weight.dtype)
    W0e = jnp.kron(eyeN, W[:, 0, :])
    W1e = jnp.kron(eyeN, W[:, 1, :])
    W2e = jnp.kron(eyeN, W[:, 2, :])
    WA = W0e - W2e
    WB = W1e
    WC = 2.0 * W2e
    bias_row = jnp.tile(bias, N).reshape(1, N * Fout)

    bm = 256
    C = N * Fin
    Co = N * Fout

    x1 = pl.pallas_call(
        _pass1_kernel,
        grid=(M // bm,),
        in_specs=[
            pl.BlockSpec((bm, M), lambda i: (i, 0)),
            pl.BlockSpec((M, C), lambda i: (0, 0)),
        ],
        out_specs=pl.BlockSpec((bm, C), lambda i: (i, 0)),
        out_shape=jax.ShapeDtypeStruct((M, C), jnp.float32),
    )(L, x0)

    out_flat = pl.pallas_call(
        _pass2_kernel,
        grid=(M // bm,),
        in_specs=[
            pl.BlockSpec((bm, M), lambda i: (i, 0)),
            pl.BlockSpec((M, C), lambda i: (0, 0)),
            pl.BlockSpec((bm, C), lambda i: (i, 0)),
            pl.BlockSpec((bm, C), lambda i: (i, 0)),
            pl.BlockSpec((C, Co), lambda i: (0, 0)),
            pl.BlockSpec((C, Co), lambda i: (0, 0)),
            pl.BlockSpec((C, Co), lambda i: (0, 0)),
            pl.BlockSpec((1, Co), lambda i: (0, 0)),
        ],
        out_specs=pl.BlockSpec((bm, Co), lambda i: (i, 0)),
        out_shape=jax.ShapeDtypeStruct((M, Co), jnp.float32),
    )(L, x1, x0, x1, WA, WB, WC, bias_row)

    return out_flat.reshape(M, N, Fout).transpose(1, 0, 2)
